# Initial kernel scaffold; baseline (speedup 1.0000x reference)
import jax
import jax.numpy as jnp
from jax.experimental import pallas as pl


def kernel(indices_list, f0, f1, f2):
    raise NotImplementedError("write your pallas kernel here")


# speedup vs baseline: 2.4801x; 2.4801x over previous
import functools

import jax
import jax.numpy as jnp
from jax import lax
from jax.experimental import pallas as pl
from jax.experimental.pallas import tpu as pltpu
from jax.experimental.pallas import tpu_sc as plsc

RANK = 16
LANES = 16
NUM_WORKERS = 32
CHUNK = 2000


def _tatd_kernel(nnz):
    num_chunks = nnz // CHUNK
    max_chunks_per_worker = (num_chunks + NUM_WORKERS - 1) // NUM_WORKERS
    groups = CHUNK // LANES

    mesh = plsc.VectorSubcoreMesh(core_axis_name="c", subcore_axis_name="s")

    @functools.partial(
        pl.kernel,
        mesh=mesh,
        compiler_params=pltpu.CompilerParams(
            needs_layout_passes=False, use_tc_tiling_on_sc=False),
        out_type=jax.ShapeDtypeStruct((nnz,), jnp.float32),
        scratch_types=[
            pltpu.VMEM((CHUNK,), jnp.int32),
            pltpu.VMEM((CHUNK,), jnp.int32),
            pltpu.VMEM((CHUNK,), jnp.int32),
            pltpu.VMEM((CHUNK, RANK), jnp.float32),
            pltpu.VMEM((CHUNK, RANK), jnp.float32),
            pltpu.VMEM((CHUNK, RANK), jnp.float32),
            pltpu.VMEM((CHUNK,), jnp.float32),
            pltpu.SemaphoreType.DMA,
        ],
    )
    def k(i0_hbm, i1_hbm, i2_hbm, f0_hbm, f1_hbm, f2_hbm, out_hbm,
          idx0_v, idx1_v, idx2_v, rows0_v, rows1_v, rows2_v, out_v, sem):
        wid = lax.axis_index("s") * 2 + lax.axis_index("c")

        def do_chunk(chunk_id):
            base = chunk_id * CHUNK
            pltpu.sync_copy(i0_hbm.at[pl.ds(base, CHUNK)], idx0_v)
            pltpu.sync_copy(i1_hbm.at[pl.ds(base, CHUNK)], idx1_v)
            pltpu.sync_copy(i2_hbm.at[pl.ds(base, CHUNK)], idx2_v)
            c0 = pltpu.async_copy(f0_hbm.at[idx0_v], rows0_v, sem)
            c1 = pltpu.async_copy(f1_hbm.at[idx1_v], rows1_v, sem)
            c2 = pltpu.async_copy(f2_hbm.at[idx2_v], rows2_v, sem)
            c0.wait()
            c1.wait()
            c2.wait()

            lane = lax.iota(jnp.int32, LANES)

            def group_body(g, _):
                row_ids = g * LANES + lane
                acc = jnp.zeros((LANES,), jnp.float32)
                for r in range(RANK):
                    col = jnp.full((LANES,), r, jnp.int32)
                    v0 = plsc.load_gather(rows0_v, [row_ids, col])
                    v1 = plsc.load_gather(rows1_v, [row_ids, col])
                    v2 = plsc.load_gather(rows2_v, [row_ids, col])
                    acc = acc + v0 * v1 * v2
                out_v[pl.ds(g * LANES, LANES)] = acc
                return 0

            lax.fori_loop(0, groups, group_body, 0)
            pltpu.sync_copy(out_v, out_hbm.at[pl.ds(base, CHUNK)])

        def chunk_loop(kk, _):
            chunk_id = kk * NUM_WORKERS + wid

            @pl.when(chunk_id < num_chunks)
            def _():
                do_chunk(chunk_id)

            return 0

        lax.fori_loop(0, max_chunks_per_worker, chunk_loop, 0)

    return k


def kernel(indices_list, f0, f1, f2):
    nnz = indices_list.shape[1]
    idx = indices_list.astype(jnp.int32)
    i0, i1, i2 = idx[0], idx[1], idx[2]
    return _tatd_kernel(nnz)(i0, i1, i2, f0, f1, f2)


# speedup vs baseline: 2.5289x; 1.0197x over previous
import functools

import jax
import jax.numpy as jnp
from jax import lax
from jax.experimental import pallas as pl
from jax.experimental.pallas import tpu as pltpu
from jax.experimental.pallas import tpu_sc as plsc

RANK = 16
LANES = 16
NUM_WORKERS = 32
CHUNK = 2000


def _tatd_kernel(nnz, ndim):
    num_chunks = nnz // CHUNK
    max_chunks_per_worker = (num_chunks + NUM_WORKERS - 1) // NUM_WORKERS
    groups = CHUNK // LANES

    mesh = plsc.VectorSubcoreMesh(core_axis_name="c", subcore_axis_name="s")

    @functools.partial(
        pl.kernel,
        mesh=mesh,
        compiler_params=pltpu.CompilerParams(
            needs_layout_passes=False, use_tc_tiling_on_sc=False),
        out_type=jax.ShapeDtypeStruct((nnz,), jnp.float32),
        scratch_types=[
            pltpu.VMEM((CHUNK,), jnp.int32),
            pltpu.VMEM((CHUNK,), jnp.int32),
            pltpu.VMEM((CHUNK,), jnp.int32),
            pltpu.VMEM((CHUNK, RANK), jnp.float32),
            pltpu.VMEM((CHUNK, RANK), jnp.float32),
            pltpu.VMEM((CHUNK, RANK), jnp.float32),
            pltpu.VMEM((CHUNK,), jnp.float32),
            pltpu.SemaphoreType.DMA,
        ],
    )
    def k(i0_hbm, i1_hbm, i2_hbm, f0_hbm, f1_hbm, f2_hbm, out_hbm,
          idx0_v, idx1_v, idx2_v, rows0_v, rows1_v, rows2_v, out_v, sem):
        wid = lax.axis_index("s") * 2 + lax.axis_index("c")
        f0t, f1t, f2t = f0_hbm, f1_hbm, f2_hbm
        lane = lax.iota(jnp.int32, LANES)

        def chunk_body(chunk_id):
            base = chunk_id * CHUNK
            pltpu.sync_copy(i0_hbm.at[pl.ds(base, CHUNK)], idx0_v)
            pltpu.sync_copy(i1_hbm.at[pl.ds(base, CHUNK)], idx1_v)
            pltpu.sync_copy(i2_hbm.at[pl.ds(base, CHUNK)], idx2_v)
            c0 = pltpu.async_copy(f0t.at[idx0_v], rows0_v, sem)
            c1 = pltpu.async_copy(f1t.at[idx1_v], rows1_v, sem)
            c2 = pltpu.async_copy(f2t.at[idx2_v], rows2_v, sem)
            c0.wait()
            c1.wait()
            c2.wait()

            @plsc.parallel_loop(0, groups)
            def group_body(g):
                row_ids = g * LANES + lane
                acc = jnp.zeros((LANES,), jnp.float32)
                for r in range(RANK):
                    col = jnp.full((LANES,), r, jnp.int32)
                    v0 = plsc.load_gather(rows0_v, [row_ids, col])
                    v1 = plsc.load_gather(rows1_v, [row_ids, col])
                    v2 = plsc.load_gather(rows2_v, [row_ids, col])
                    acc = acc + v0 * v1 * v2
                out_v[pl.ds(g * LANES, LANES)] = acc

            pltpu.sync_copy(out_v, out_hbm.at[pl.ds(base, CHUNK)])

        def chunk_loop(kk, _):
            chunk_id = kk * NUM_WORKERS + wid

            @pl.when(chunk_id < num_chunks)
            def _():
                chunk_body(chunk_id)

            return 0

        lax.fori_loop(0, max_chunks_per_worker, chunk_loop, 0)

    return k


def kernel(indices_list, f0, f1, f2):
    nnz = indices_list.shape[1]
    ndim = f0.shape[0]
    idx = indices_list.astype(jnp.int32)
    i0, i1, i2 = idx[0], idx[1], idx[2]
    return _tatd_kernel(nnz, ndim)(i0, i1, i2, f0, f1, f2)
